# baseline (speedup 1.0000x reference)
import jax
import jax.numpy as jnp
from jax.experimental import pallas as pl
from jax.experimental.pallas import tpu as pltpu

_CBLK = 128
_SBLK = 9216


def _pos_kernel(col_ref, row_ref, out_ref):
    j = pl.program_id(1)
    nx = pl.num_programs(1) // 2
    cblk = out_ref.shape[1]
    sblk = out_ref.shape[2]
    w = col_ref.shape[0]
    nh = sblk // w

    @pl.when(j < nx)
    def _x_part():
        xt = col_ref[...].T
        out_ref[...] = jnp.broadcast_to(
            xt[None, :, None, :], (1, cblk, nh, w)
        ).reshape(1, cblk, sblk)

    @pl.when(j >= nx)
    def _y_part():
        yt = row_ref[...].T
        out_ref[...] = jnp.broadcast_to(
            yt[None, :, :, None], (1, cblk, nh, w)
        ).reshape(1, cblk, sblk)


def kernel(pixel_values, row_weight, col_weight):
    batch = pixel_values.shape[0]
    height, width = pixel_values.shape[-2], pixel_values.shape[-1]
    embed_dim = row_weight.shape[1]
    cblk = _CBLK
    sblk = _SBLK
    nx = embed_dim // cblk
    nh = sblk // width
    ns = (height * width) // sblk

    out = pl.pallas_call(
        _pos_kernel,
        grid=(batch, 2 * nx, ns),
        in_specs=[
            pl.BlockSpec((width, cblk), lambda b, j, s: (0, j % nx)),
            pl.BlockSpec((nh, cblk), lambda b, j, s: (s, j % nx)),
        ],
        out_specs=pl.BlockSpec(
            (1, cblk, sblk), lambda b, j, s: (b, j, s)
        ),
        out_shape=jax.ShapeDtypeStruct(
            (batch, 2 * embed_dim, height * width), jnp.float32
        ),
    )(col_weight[:width, :], row_weight[:height, :])
    return out.reshape(batch, 2 * embed_dim, height, width)
